# baseline (speedup 1.0000x reference)
import jax
import jax.numpy as jnp
from jax import lax
from jax.experimental import pallas as pl
from jax.experimental.pallas import tpu as pltpu
from jax.experimental.pallas import tpu_sc as plsc

N_NODES = 10000
D = 128
N_EDGES = 320000

NC = 2
NS = 16
NW = NC * NS
EPW = N_EDGES // NW
C = 100
NB = 3
Q = 5
SCH = 20
SEG = 640
LAST_SEG = N_NODES - 15 * SEG
ZR = 80

_sc_mesh = plsc.VectorSubcoreMesh(core_axis_name="c", subcore_axis_name="s")


def _agg_body(ei_hbm, feat_hbm, zeros_hbm, out_hbm,
              sidx, didx, rows0, rows1, rows2,
              acc, gs0, gs1, gs2, ss0, ss1, ss2):
    cid = lax.axis_index("c")
    sid = lax.axis_index("s")
    wid = sid * NC + cid
    bufs = (rows0, rows1, rows2)
    gsems = (gs0, gs1, gs2)
    ssems = (ss0, ss1, ss2)

    pltpu.sync_copy(zeros_hbm, rows0)

    @pl.when(sid < NS - 1)
    def _():
        for k in range(SEG // ZR):
            pltpu.sync_copy(rows0.at[pl.ds(0, ZR)],
                            acc.at[pl.ds(sid * SEG + k * ZR, ZR)])

    @pl.when(sid == NS - 1)
    def _():
        for k in range(LAST_SEG // ZR):
            pltpu.sync_copy(rows0.at[pl.ds(0, ZR)],
                            acc.at[pl.ds(15 * SEG + k * ZR, ZR)])

    plsc.subcore_barrier()

    def _gather(c, b):
        pltpu.async_copy(feat_hbm.at[sidx.at[c]], bufs[b], gsems[b])

    def _gwait(b):
        pltpu.make_async_copy(feat_hbm.at[sidx.at[0]], bufs[b], gsems[b]).wait()

    def _scatter(c, b):
        pltpu.async_copy(bufs[b], acc.at[didx.at[c]], ssems[b], add=True)

    def _swait(b):
        pltpu.make_async_copy(bufs[b], acc.at[didx.at[0]], ssems[b]).wait()

    for q in range(Q):
        pltpu.sync_copy(ei_hbm.at[0, wid, q], sidx)
        pltpu.sync_copy(ei_hbm.at[1, wid, q], didx)

        for b in range(NB):
            _gather(b, b)

        def _quad(p, carry):
            c0 = NB * p
            for b in range(NB):
                _gwait(b)
                pltpu.sync_copy(bufs[b], acc.at[didx.at[c0 + b]], add=True)

                @pl.when(c0 + b + NB < SCH)
                def _():
                    _gather(c0 + b + NB, b)

            return carry

        lax.fori_loop(0, SCH // NB, _quad, 0)
        for c in range((SCH // NB) * NB, SCH):
            _gwait(c % NB)
            pltpu.sync_copy(bufs[c % NB], acc.at[didx.at[c]], add=True)

    plsc.subcore_barrier()

    @pl.when(sid < NS - 1)
    def _():
        pltpu.sync_copy(acc.at[pl.ds(sid * SEG, SEG)],
                        out_hbm.at[cid, pl.ds(sid * SEG, SEG)])

    @pl.when(sid == NS - 1)
    def _():
        pltpu.sync_copy(acc.at[pl.ds(15 * SEG, LAST_SEG)],
                        out_hbm.at[cid, pl.ds(15 * SEG, LAST_SEG)])


_agg = pl.kernel(
    _agg_body,
    out_type=jax.ShapeDtypeStruct((NC, N_NODES, D), jnp.float32),
    mesh=_sc_mesh,
    scratch_types=(
        [pltpu.VMEM((SCH, C), jnp.int32)] * 2
        + [pltpu.VMEM((C, D), jnp.float32)] * NB
        + [pltpu.VMEM_SHARED((N_NODES, D), jnp.float32)]
        + [pltpu.SemaphoreType.DMA] * (2 * NB)
    ),
)


def _mm1_body(feat_ref, w1_ref, o_ref):
    o_ref[...] = jnp.dot(feat_ref[...], w1_ref[...],
                         preferred_element_type=jnp.float32)


def _mm2_body(part1_ref, p_ref, w2_ref, o_ref):
    agg = p_ref[0] + p_ref[1]
    o_ref[...] = part1_ref[...] + jnp.dot(
        agg, w2_ref[...], preferred_element_type=jnp.float32)


_ROWS_BLK = 1000


def _mm1(feat, w1):
    return pl.pallas_call(
        _mm1_body,
        grid=(N_NODES // _ROWS_BLK,),
        in_specs=[
            pl.BlockSpec((_ROWS_BLK, D), lambda i: (i, 0)),
            pl.BlockSpec((D, D), lambda i: (0, 0)),
        ],
        out_specs=pl.BlockSpec((_ROWS_BLK, D), lambda i: (i, 0)),
        out_shape=jax.ShapeDtypeStruct((N_NODES, D), jnp.float32),
    )(feat, w1)


def _mm2(part1, partials, w2):
    return pl.pallas_call(
        _mm2_body,
        grid=(N_NODES // _ROWS_BLK,),
        in_specs=[
            pl.BlockSpec((_ROWS_BLK, D), lambda i: (i, 0)),
            pl.BlockSpec((NC, _ROWS_BLK, D), lambda i: (0, i, 0)),
            pl.BlockSpec((D, D), lambda i: (0, 0)),
        ],
        out_specs=pl.BlockSpec((_ROWS_BLK, D), lambda i: (i, 0)),
        out_shape=jax.ShapeDtypeStruct((N_NODES, D), jnp.float32),
    )(part1, partials, w2)


@jax.jit
def kernel(feat, edge_index, weight1, weight2):
    ei5 = edge_index.reshape(2, NW, Q, SCH, C)
    zeros = jnp.zeros((C, D), jnp.float32)
    partials = _agg(ei5, feat, zeros)
    part1 = _mm1(feat, weight1)
    return _mm2(part1, partials, weight2)


# speedup vs baseline: 1.0110x; 1.0110x over previous
import jax
import jax.numpy as jnp
from jax import lax
from jax.experimental import pallas as pl
from jax.experimental.pallas import tpu as pltpu
from jax.experimental.pallas import tpu_sc as plsc

N_NODES = 10000
D = 128
N_EDGES = 320000

NC = 2
NS = 16
NW = NC * NS
EPW = N_EDGES // NW
C = 80
NB = 4
Q = 5
SCH = 25
SEG = 640
LAST_SEG = N_NODES - 15 * SEG
ZR = 80

_sc_mesh = plsc.VectorSubcoreMesh(core_axis_name="c", subcore_axis_name="s")


def _agg_body(ei_hbm, feat_hbm, zeros_hbm, out_hbm,
              sidx, didx, rows0, rows1, rows2, rows3,
              acc, gs0, gs1, gs2, gs3, ss0, ss1, ss2, ss3):
    cid = lax.axis_index("c")
    sid = lax.axis_index("s")
    wid = sid * NC + cid
    bufs = (rows0, rows1, rows2, rows3)
    gsems = (gs0, gs1, gs2, gs3)
    ssems = (ss0, ss1, ss2, ss3)

    pltpu.sync_copy(zeros_hbm, rows0)

    @pl.when(sid < NS - 1)
    def _():
        for k in range(SEG // ZR):
            pltpu.sync_copy(rows0.at[pl.ds(0, ZR)],
                            acc.at[pl.ds(sid * SEG + k * ZR, ZR)])

    @pl.when(sid == NS - 1)
    def _():
        for k in range(LAST_SEG // ZR):
            pltpu.sync_copy(rows0.at[pl.ds(0, ZR)],
                            acc.at[pl.ds(15 * SEG + k * ZR, ZR)])

    plsc.subcore_barrier()

    def _gather(c, b):
        pltpu.async_copy(feat_hbm.at[sidx.at[c]], bufs[b], gsems[b])

    def _gwait(b):
        pltpu.make_async_copy(feat_hbm.at[sidx.at[0]], bufs[b], gsems[b]).wait()

    def _scatter(c, b):
        pltpu.async_copy(bufs[b], acc.at[didx.at[c]], ssems[b], add=True)

    def _swait(b):
        pltpu.make_async_copy(bufs[b], acc.at[didx.at[0]], ssems[b]).wait()

    for q in range(Q):
        pltpu.sync_copy(ei_hbm.at[0, wid, q], sidx)
        pltpu.sync_copy(ei_hbm.at[1, wid, q], didx)

        for b in range(NB):
            _gather(b, b)

        def _quad(p, carry):
            c0 = NB * p
            for b in range(NB):
                _gwait(b)
                pltpu.sync_copy(bufs[b], acc.at[didx.at[c0 + b]], add=True)

                @pl.when(c0 + b + NB < SCH)
                def _():
                    _gather(c0 + b + NB, b)

            return carry

        lax.fori_loop(0, SCH // NB, _quad, 0)
        for c in range((SCH // NB) * NB, SCH):
            _gwait(c % NB)
            pltpu.sync_copy(bufs[c % NB], acc.at[didx.at[c]], add=True)

    plsc.subcore_barrier()

    @pl.when(sid < NS - 1)
    def _():
        pltpu.sync_copy(acc.at[pl.ds(sid * SEG, SEG)],
                        out_hbm.at[cid, pl.ds(sid * SEG, SEG)])

    @pl.when(sid == NS - 1)
    def _():
        pltpu.sync_copy(acc.at[pl.ds(15 * SEG, LAST_SEG)],
                        out_hbm.at[cid, pl.ds(15 * SEG, LAST_SEG)])


_agg = pl.kernel(
    _agg_body,
    out_type=jax.ShapeDtypeStruct((NC, N_NODES, D), jnp.float32),
    mesh=_sc_mesh,
    scratch_types=(
        [pltpu.VMEM((SCH, C), jnp.int32)] * 2
        + [pltpu.VMEM((C, D), jnp.float32)] * NB
        + [pltpu.VMEM_SHARED((N_NODES, D), jnp.float32)]
        + [pltpu.SemaphoreType.DMA] * (2 * NB)
    ),
)


def _mm1_body(feat_ref, w1_ref, o_ref):
    o_ref[...] = jnp.dot(feat_ref[...], w1_ref[...],
                         preferred_element_type=jnp.float32)


def _mm2_body(part1_ref, p_ref, w2_ref, o_ref):
    agg = p_ref[0] + p_ref[1]
    o_ref[...] = part1_ref[...] + jnp.dot(
        agg, w2_ref[...], preferred_element_type=jnp.float32)


_ROWS_BLK = 1000


def _mm1(feat, w1):
    return pl.pallas_call(
        _mm1_body,
        grid=(N_NODES // _ROWS_BLK,),
        in_specs=[
            pl.BlockSpec((_ROWS_BLK, D), lambda i: (i, 0)),
            pl.BlockSpec((D, D), lambda i: (0, 0)),
        ],
        out_specs=pl.BlockSpec((_ROWS_BLK, D), lambda i: (i, 0)),
        out_shape=jax.ShapeDtypeStruct((N_NODES, D), jnp.float32),
    )(feat, w1)


def _mm2(part1, partials, w2):
    return pl.pallas_call(
        _mm2_body,
        grid=(N_NODES // _ROWS_BLK,),
        in_specs=[
            pl.BlockSpec((_ROWS_BLK, D), lambda i: (i, 0)),
            pl.BlockSpec((NC, _ROWS_BLK, D), lambda i: (0, i, 0)),
            pl.BlockSpec((D, D), lambda i: (0, 0)),
        ],
        out_specs=pl.BlockSpec((_ROWS_BLK, D), lambda i: (i, 0)),
        out_shape=jax.ShapeDtypeStruct((N_NODES, D), jnp.float32),
    )(part1, partials, w2)


@jax.jit
def kernel(feat, edge_index, weight1, weight2):
    ei5 = edge_index.reshape(2, NW, Q, SCH, C)
    zeros = jnp.zeros((C, D), jnp.float32)
    partials = _agg(ei5, feat, zeros)
    part1 = _mm1(feat, weight1)
    return _mm2(part1, partials, weight2)


# speedup vs baseline: 1.0428x; 1.0314x over previous
import jax
import jax.numpy as jnp
from jax import lax
from jax.experimental import pallas as pl
from jax.experimental.pallas import tpu as pltpu
from jax.experimental.pallas import tpu_sc as plsc

N_NODES = 10000
D = 128
N_EDGES = 320000

NC = 2
NS = 16
NW = NC * NS
BLK = 128
NBLK = N_EDGES // BLK
BPW = NBLK // NW
C = 64
NB = 4
STAGES = (16, 16, 16, 16, 14)
SMAX = 16
SEG = 640
LAST_SEG = N_NODES - 15 * SEG
ZR = 40

_sc_mesh = plsc.VectorSubcoreMesh(core_axis_name="c", subcore_axis_name="s")


def _agg_body(ei_hbm, feat_hbm, zeros_hbm, out_hbm,
              sidx, didx, rows0, rows1, rows2, rows3,
              acc, gs0, gs1, gs2, gs3, ss0, ss1, ss2, ss3):
    cid = lax.axis_index("c")
    sid = lax.axis_index("s")
    wid = sid * NC + cid
    bufs = (rows0, rows1, rows2, rows3)
    gsems = (gs0, gs1, gs2, gs3)
    ssems = (ss0, ss1, ss2, ss3)

    pltpu.sync_copy(zeros_hbm, rows0)

    @pl.when(sid < NS - 1)
    def _():
        for k in range(SEG // ZR):
            pltpu.sync_copy(rows0.at[pl.ds(0, ZR)],
                            acc.at[pl.ds(sid * SEG + k * ZR, ZR)])

    @pl.when(sid == NS - 1)
    def _():
        for k in range(LAST_SEG // ZR):
            pltpu.sync_copy(rows0.at[pl.ds(0, ZR)],
                            acc.at[pl.ds(15 * SEG + k * ZR, ZR)])

    plsc.subcore_barrier()

    def _gather(c, b):
        pltpu.async_copy(feat_hbm.at[sidx.at[pl.ds(c * C, C)]], bufs[b],
                         gsems[b])

    def _gwait(b):
        pltpu.make_async_copy(feat_hbm.at[sidx.at[pl.ds(0, C)]], bufs[b],
                              gsems[b]).wait()

    base = (BPW * wid + jnp.minimum(wid, 4)) * BLK

    off = 0
    for nblk in STAGES:
        n = nblk * BLK
        e0 = pl.multiple_of(base + off, BLK)
        pltpu.sync_copy(ei_hbm.at[0, pl.ds(e0, n)], sidx.at[pl.ds(0, n)])
        pltpu.sync_copy(ei_hbm.at[1, pl.ds(e0, n)], didx.at[pl.ds(0, n)])
        off += n

        nch = (nblk * BLK) // C
        for b in range(NB):
            _gather(b, b)

        def _quad(p, carry):
            c0 = NB * p
            for b in range(NB):
                _gwait(b)
                pltpu.sync_copy(bufs[b],
                                acc.at[didx.at[pl.ds((c0 + b) * C, C)]],
                                add=True)

                @pl.when(c0 + b + NB < nch)
                def _():
                    _gather(c0 + b + NB, b)

            return carry

        lax.fori_loop(0, nch // NB, _quad, 0)

    @pl.when(wid < 4)
    def _():
        e0 = pl.multiple_of(base + BPW * BLK, BLK)
        pltpu.sync_copy(ei_hbm.at[0, pl.ds(e0, BLK)], sidx.at[pl.ds(0, BLK)])
        pltpu.sync_copy(ei_hbm.at[1, pl.ds(e0, BLK)], didx.at[pl.ds(0, BLK)])
        for c in range(BLK // C):
            pltpu.sync_copy(feat_hbm.at[sidx.at[pl.ds(c * C, C)]], rows0)
            pltpu.sync_copy(rows0, acc.at[didx.at[pl.ds(c * C, C)]], add=True)

    plsc.subcore_barrier()

    @pl.when(sid < NS - 1)
    def _():
        pltpu.sync_copy(acc.at[pl.ds(sid * SEG, SEG)],
                        out_hbm.at[cid, pl.ds(sid * SEG, SEG)])

    @pl.when(sid == NS - 1)
    def _():
        pltpu.sync_copy(acc.at[pl.ds(15 * SEG, LAST_SEG)],
                        out_hbm.at[cid, pl.ds(15 * SEG, LAST_SEG)])


_agg = pl.kernel(
    _agg_body,
    out_type=jax.ShapeDtypeStruct((NC, N_NODES, D), jnp.float32),
    mesh=_sc_mesh,
    scratch_types=(
        [pltpu.VMEM((SMAX * BLK,), jnp.int32)] * 2
        + [pltpu.VMEM((C, D), jnp.float32)] * NB
        + [pltpu.VMEM_SHARED((N_NODES, D), jnp.float32)]
        + [pltpu.SemaphoreType.DMA] * (2 * NB)
    ),
)


def _mm1_body(feat_ref, w1_ref, o_ref):
    o_ref[...] = jnp.dot(feat_ref[...], w1_ref[...],
                         preferred_element_type=jnp.float32)


def _mm2_body(part1_ref, p_ref, w2_ref, o_ref):
    agg = p_ref[0] + p_ref[1]
    o_ref[...] = part1_ref[...] + jnp.dot(
        agg, w2_ref[...], preferred_element_type=jnp.float32)


_ROWS_BLK = 1000


def _mm1(feat, w1):
    return pl.pallas_call(
        _mm1_body,
        grid=(N_NODES // _ROWS_BLK,),
        in_specs=[
            pl.BlockSpec((_ROWS_BLK, D), lambda i: (i, 0)),
            pl.BlockSpec((D, D), lambda i: (0, 0)),
        ],
        out_specs=pl.BlockSpec((_ROWS_BLK, D), lambda i: (i, 0)),
        out_shape=jax.ShapeDtypeStruct((N_NODES, D), jnp.float32),
    )(feat, w1)


def _mm2(part1, partials, w2):
    return pl.pallas_call(
        _mm2_body,
        grid=(N_NODES // _ROWS_BLK,),
        in_specs=[
            pl.BlockSpec((_ROWS_BLK, D), lambda i: (i, 0)),
            pl.BlockSpec((NC, _ROWS_BLK, D), lambda i: (0, i, 0)),
            pl.BlockSpec((D, D), lambda i: (0, 0)),
        ],
        out_specs=pl.BlockSpec((_ROWS_BLK, D), lambda i: (i, 0)),
        out_shape=jax.ShapeDtypeStruct((N_NODES, D), jnp.float32),
    )(part1, partials, w2)


@jax.jit
def kernel(feat, edge_index, weight1, weight2):
    zeros = jnp.zeros((C, D), jnp.float32)
    partials = _agg(edge_index, feat, zeros)
    part1 = _mm1(feat, weight1)
    return _mm2(part1, partials, weight2)


# speedup vs baseline: 1.0702x; 1.0263x over previous
import jax
import jax.numpy as jnp
from jax import lax
from jax.experimental import pallas as pl
from jax.experimental.pallas import tpu as pltpu
from jax.experimental.pallas import tpu_sc as plsc

N_NODES = 10000
D = 128
N_EDGES = 320000

NC = 2
NS = 16
NW = NC * NS
BLK = 128
NBLK = N_EDGES // BLK
BPW = NBLK // NW
C = 64
NB = 4
STAGES = (16, 16, 16, 16, 14)
SMAX = 16
SEG = 640
LAST_SEG = N_NODES - 15 * SEG
ZR = 40

_sc_mesh = plsc.VectorSubcoreMesh(core_axis_name="c", subcore_axis_name="s")


def _agg_body(ei_hbm, feat_hbm, zeros_hbm, out_hbm,
              sidx, didx, rows0, rows1, rows2, rows3,
              acc, gs0, gs1, gs2, gs3, ss0, ss1, ss2, ss3):
    cid = lax.axis_index("c")
    sid = lax.axis_index("s")
    wid = sid * NC + cid
    bufs = (rows0, rows1, rows2, rows3)
    gsems = (gs0, gs1, gs2, gs3)
    ssems = (ss0, ss1, ss2, ss3)

    pltpu.sync_copy(zeros_hbm, rows0)

    @pl.when(sid < NS - 1)
    def _():
        for k in range(SEG // ZR):
            pltpu.sync_copy(rows0.at[pl.ds(0, ZR)],
                            acc.at[pl.ds(sid * SEG + k * ZR, ZR)])

    @pl.when(sid == NS - 1)
    def _():
        for k in range(LAST_SEG // ZR):
            pltpu.sync_copy(rows0.at[pl.ds(0, ZR)],
                            acc.at[pl.ds(15 * SEG + k * ZR, ZR)])

    def _gather(c, b):
        pltpu.async_copy(feat_hbm.at[sidx.at[pl.ds(c * C, C)]], bufs[b],
                         gsems[b])

    def _gwait(b):
        pltpu.make_async_copy(feat_hbm.at[sidx.at[pl.ds(0, C)]], bufs[b],
                              gsems[b]).wait()

    base = (BPW * wid + jnp.minimum(wid, 4)) * BLK

    off = 0
    for qi, nblk in enumerate(STAGES):
        n = nblk * BLK
        e0 = pl.multiple_of(base + off, BLK)
        pltpu.sync_copy(ei_hbm.at[0, pl.ds(e0, n)], sidx.at[pl.ds(0, n)])
        pltpu.sync_copy(ei_hbm.at[1, pl.ds(e0, n)], didx.at[pl.ds(0, n)])
        off += n

        nch = (nblk * BLK) // C
        for b in range(NB):
            _gather(b, b)
        if qi == 0:
            plsc.subcore_barrier()

        def _quad(p, carry):
            c0 = NB * p
            for b in range(NB):
                _gwait(b)
                pltpu.sync_copy(bufs[b],
                                acc.at[didx.at[pl.ds((c0 + b) * C, C)]],
                                add=True)

                @pl.when(c0 + b + NB < nch)
                def _():
                    _gather(c0 + b + NB, b)

            return carry

        lax.fori_loop(0, nch // NB, _quad, 0)

    @pl.when(wid < 4)
    def _():
        e0 = pl.multiple_of(base + BPW * BLK, BLK)
        pltpu.sync_copy(ei_hbm.at[0, pl.ds(e0, BLK)], sidx.at[pl.ds(0, BLK)])
        pltpu.sync_copy(ei_hbm.at[1, pl.ds(e0, BLK)], didx.at[pl.ds(0, BLK)])
        for c in range(BLK // C):
            _gather(c, c)
        for c in range(BLK // C):
            _gwait(c)
            pltpu.sync_copy(bufs[c], acc.at[didx.at[pl.ds(c * C, C)]],
                            add=True)

    plsc.subcore_barrier()

    @pl.when(sid < NS - 1)
    def _():
        pltpu.sync_copy(acc.at[pl.ds(sid * SEG, SEG)],
                        out_hbm.at[cid, pl.ds(sid * SEG, SEG)])

    @pl.when(sid == NS - 1)
    def _():
        pltpu.sync_copy(acc.at[pl.ds(15 * SEG, LAST_SEG)],
                        out_hbm.at[cid, pl.ds(15 * SEG, LAST_SEG)])


_agg = pl.kernel(
    _agg_body,
    out_type=jax.ShapeDtypeStruct((NC, N_NODES, D), jnp.float32),
    mesh=_sc_mesh,
    scratch_types=(
        [pltpu.VMEM((SMAX * BLK,), jnp.int32)] * 2
        + [pltpu.VMEM((C, D), jnp.float32)] * NB
        + [pltpu.VMEM_SHARED((N_NODES, D), jnp.float32)]
        + [pltpu.SemaphoreType.DMA] * (2 * NB)
    ),
)


def _mm1_body(feat_ref, w1_ref, o_ref):
    o_ref[...] = jnp.dot(feat_ref[...], w1_ref[...],
                         preferred_element_type=jnp.float32)


def _mm2_body(part1_ref, p_ref, w2_ref, o_ref):
    agg = p_ref[0] + p_ref[1]
    o_ref[...] = part1_ref[...] + jnp.dot(
        agg, w2_ref[...], preferred_element_type=jnp.float32)


_ROWS_BLK = 2000


def _mm1(feat, w1):
    return pl.pallas_call(
        _mm1_body,
        grid=(N_NODES // _ROWS_BLK,),
        in_specs=[
            pl.BlockSpec((_ROWS_BLK, D), lambda i: (i, 0)),
            pl.BlockSpec((D, D), lambda i: (0, 0)),
        ],
        out_specs=pl.BlockSpec((_ROWS_BLK, D), lambda i: (i, 0)),
        out_shape=jax.ShapeDtypeStruct((N_NODES, D), jnp.float32),
    )(feat, w1)


def _mm2(part1, partials, w2):
    return pl.pallas_call(
        _mm2_body,
        grid=(N_NODES // _ROWS_BLK,),
        in_specs=[
            pl.BlockSpec((_ROWS_BLK, D), lambda i: (i, 0)),
            pl.BlockSpec((NC, _ROWS_BLK, D), lambda i: (0, i, 0)),
            pl.BlockSpec((D, D), lambda i: (0, 0)),
        ],
        out_specs=pl.BlockSpec((_ROWS_BLK, D), lambda i: (i, 0)),
        out_shape=jax.ShapeDtypeStruct((N_NODES, D), jnp.float32),
    )(part1, partials, w2)


@jax.jit
def kernel(feat, edge_index, weight1, weight2):
    zeros = jnp.zeros((C, D), jnp.float32)
    partials = _agg(edge_index, feat, zeros)
    part1 = _mm1(feat, weight1)
    return _mm2(part1, partials, weight2)


# speedup vs baseline: 1.1152x; 1.0421x over previous
import jax
import jax.numpy as jnp
from jax import lax
from jax.experimental import pallas as pl
from jax.experimental.pallas import tpu as pltpu
from jax.experimental.pallas import tpu_sc as plsc

N_NODES = 10000
D = 128
N_EDGES = 320000

NC = 2
NS = 16
NW = NC * NS
BLK = 128
NBLK = N_EDGES // BLK
BPW = NBLK // NW
EW = BPW * BLK
C = 64
NCH = EW // C
NB = 3
SEG = 640
LAST_SEG = N_NODES - 15 * SEG
ZR = 40

_sc_mesh = plsc.VectorSubcoreMesh(core_axis_name="c", subcore_axis_name="s")


def _agg_body(ei_hbm, feat_hbm, zeros_hbm, out_hbm,
              sidx, didx, rows0, rows1, rows2,
              acc, gs0, gs1, gs2):
    cid = lax.axis_index("c")
    sid = lax.axis_index("s")
    wid = sid * NC + cid
    bufs = (rows0, rows1, rows2)
    gsems = (gs0, gs1, gs2)

    base = pl.multiple_of((BPW * wid + jnp.minimum(wid, 4)) * BLK, BLK)

    cp_s = pltpu.async_copy(ei_hbm.at[0, pl.ds(base, EW)], sidx, gs0)
    cp_d = pltpu.async_copy(ei_hbm.at[1, pl.ds(base, EW)], didx, gs1)

    pltpu.sync_copy(zeros_hbm, rows0)

    @pl.when(sid < NS - 1)
    def _():
        for k in range(SEG // ZR):
            pltpu.sync_copy(rows0.at[pl.ds(0, ZR)],
                            acc.at[pl.ds(sid * SEG + k * ZR, ZR)])

    @pl.when(sid == NS - 1)
    def _():
        for k in range(LAST_SEG // ZR):
            pltpu.sync_copy(rows0.at[pl.ds(0, ZR)],
                            acc.at[pl.ds(15 * SEG + k * ZR, ZR)])

    cp_s.wait()
    cp_d.wait()

    def _gather(c, b):
        pltpu.async_copy(feat_hbm.at[sidx.at[pl.ds(c * C, C)]], bufs[b],
                         gsems[b])

    def _gwait(b):
        pltpu.make_async_copy(feat_hbm.at[sidx.at[pl.ds(0, C)]], bufs[b],
                              gsems[b]).wait()

    for b in range(NB):
        _gather(b, b)
    plsc.subcore_barrier()

    def _quad(p, carry):
        c0 = NB * p
        for b in range(NB):
            _gwait(b)
            pltpu.sync_copy(bufs[b],
                            acc.at[didx.at[pl.ds((c0 + b) * C, C)]],
                            add=True)

            @pl.when(c0 + b + NB < NCH)
            def _():
                _gather(c0 + b + NB, b)

        return carry

    lax.fori_loop(0, NCH // NB, _quad, 0)

    @pl.when(wid < 4)
    def _():
        e0 = pl.multiple_of(base + EW, BLK)
        pltpu.sync_copy(ei_hbm.at[0, pl.ds(e0, BLK)], sidx.at[pl.ds(0, BLK)])
        pltpu.sync_copy(ei_hbm.at[1, pl.ds(e0, BLK)], didx.at[pl.ds(0, BLK)])
        for c in range(BLK // C):
            _gather(c, c)
        for c in range(BLK // C):
            _gwait(c)
            pltpu.sync_copy(bufs[c], acc.at[didx.at[pl.ds(c * C, C)]],
                            add=True)

    plsc.subcore_barrier()

    @pl.when(sid < NS - 1)
    def _():
        pltpu.sync_copy(acc.at[pl.ds(sid * SEG, SEG)],
                        out_hbm.at[cid, pl.ds(sid * SEG, SEG)])

    @pl.when(sid == NS - 1)
    def _():
        pltpu.sync_copy(acc.at[pl.ds(15 * SEG, LAST_SEG)],
                        out_hbm.at[cid, pl.ds(15 * SEG, LAST_SEG)])


_agg = pl.kernel(
    _agg_body,
    out_type=jax.ShapeDtypeStruct((NC, N_NODES, D), jnp.float32),
    mesh=_sc_mesh,
    scratch_types=(
        [pltpu.VMEM((EW,), jnp.int32)] * 2
        + [pltpu.VMEM((C, D), jnp.float32)] * NB
        + [pltpu.VMEM_SHARED((N_NODES, D), jnp.float32)]
        + [pltpu.SemaphoreType.DMA] * NB
    ),
)


def _mm1_body(feat_ref, w1_ref, o_ref):
    o_ref[...] = jnp.dot(feat_ref[...], w1_ref[...],
                         preferred_element_type=jnp.float32)


def _mm2_body(part1_ref, p_ref, w2_ref, o_ref):
    agg = p_ref[0] + p_ref[1]
    o_ref[...] = part1_ref[...] + jnp.dot(
        agg, w2_ref[...], preferred_element_type=jnp.float32)


_ROWS_BLK = 2000


def _mm1(feat, w1):
    return pl.pallas_call(
        _mm1_body,
        grid=(N_NODES // _ROWS_BLK,),
        in_specs=[
            pl.BlockSpec((_ROWS_BLK, D), lambda i: (i, 0)),
            pl.BlockSpec((D, D), lambda i: (0, 0)),
        ],
        out_specs=pl.BlockSpec((_ROWS_BLK, D), lambda i: (i, 0)),
        out_shape=jax.ShapeDtypeStruct((N_NODES, D), jnp.float32),
    )(feat, w1)


def _mm2(part1, partials, w2):
    return pl.pallas_call(
        _mm2_body,
        grid=(N_NODES // _ROWS_BLK,),
        in_specs=[
            pl.BlockSpec((_ROWS_BLK, D), lambda i: (i, 0)),
            pl.BlockSpec((NC, _ROWS_BLK, D), lambda i: (0, i, 0)),
            pl.BlockSpec((D, D), lambda i: (0, 0)),
        ],
        out_specs=pl.BlockSpec((_ROWS_BLK, D), lambda i: (i, 0)),
        out_shape=jax.ShapeDtypeStruct((N_NODES, D), jnp.float32),
    )(part1, partials, w2)


@jax.jit
def kernel(feat, edge_index, weight1, weight2):
    zeros = jnp.zeros((C, D), jnp.float32)
    partials = _agg(edge_index, feat, zeros)
    part1 = _mm1(feat, weight1)
    return _mm2(part1, partials, weight2)


# speedup vs baseline: 1.1235x; 1.0074x over previous
import jax
import jax.numpy as jnp
from jax import lax
from jax.experimental import pallas as pl
from jax.experimental.pallas import tpu as pltpu
from jax.experimental.pallas import tpu_sc as plsc

N_NODES = 10000
D = 128
N_EDGES = 320000

NC = 2
NS = 16
NW = NC * NS
BLK = 128
NBLK = N_EDGES // BLK
BPW = NBLK // NW
EW = BPW * BLK
C = 64
NCH = EW // C
NB = 3
SEG = 640
LAST_SEG = N_NODES - 15 * SEG
ZR = 40

_sc_mesh = plsc.VectorSubcoreMesh(core_axis_name="c", subcore_axis_name="s")


def _agg_body(ei_hbm, feat_hbm, zeros_hbm, out_hbm,
              sidx, didx, rows0, rows1, rows2,
              acc, gs0, gs1, gs2, zsem):
    cid = lax.axis_index("c")
    sid = lax.axis_index("s")
    wid = sid * NC + cid
    bufs = (rows0, rows1, rows2)
    gsems = (gs0, gs1, gs2)

    base = pl.multiple_of((BPW * wid + jnp.minimum(wid, 4)) * BLK, BLK)

    cp_s = pltpu.async_copy(ei_hbm.at[0, pl.ds(base, EW)], sidx, gs0)
    cp_d = pltpu.async_copy(ei_hbm.at[1, pl.ds(base, EW)], didx, gs1)

    pltpu.sync_copy(zeros_hbm, rows0)

    @pl.when(sid < NS - 1)
    def _():
        for k in range(SEG // ZR):
            pltpu.async_copy(rows0.at[pl.ds(0, ZR)],
                             acc.at[pl.ds(sid * SEG + k * ZR, ZR)], zsem)
        for k in range(SEG // ZR):
            pltpu.make_async_copy(rows0.at[pl.ds(0, ZR)],
                                  acc.at[pl.ds(sid * SEG + k * ZR, ZR)],
                                  zsem).wait()

    @pl.when(sid == NS - 1)
    def _():
        for k in range(LAST_SEG // ZR):
            pltpu.async_copy(rows0.at[pl.ds(0, ZR)],
                             acc.at[pl.ds(15 * SEG + k * ZR, ZR)], zsem)
        for k in range(LAST_SEG // ZR):
            pltpu.make_async_copy(rows0.at[pl.ds(0, ZR)],
                                  acc.at[pl.ds(15 * SEG + k * ZR, ZR)],
                                  zsem).wait()

    cp_s.wait()
    cp_d.wait()

    def _gather(c, b):
        pltpu.async_copy(feat_hbm.at[sidx.at[pl.ds(c * C, C)]], bufs[b],
                         gsems[b])

    def _gwait(b):
        pltpu.make_async_copy(feat_hbm.at[sidx.at[pl.ds(0, C)]], bufs[b],
                              gsems[b]).wait()

    for b in range(NB):
        _gather(b, b)
    plsc.subcore_barrier()

    def _quad(p, carry):
        c0 = NB * p
        for b in range(NB):
            _gwait(b)
            pltpu.sync_copy(bufs[b],
                            acc.at[didx.at[pl.ds((c0 + b) * C, C)]],
                            add=True)

            @pl.when(c0 + b + NB < NCH)
            def _():
                _gather(c0 + b + NB, b)

        return carry

    lax.fori_loop(0, NCH // NB, _quad, 0)

    @pl.when(wid < 4)
    def _():
        e0 = pl.multiple_of(base + EW, BLK)
        pltpu.sync_copy(ei_hbm.at[0, pl.ds(e0, BLK)], sidx.at[pl.ds(0, BLK)])
        pltpu.sync_copy(ei_hbm.at[1, pl.ds(e0, BLK)], didx.at[pl.ds(0, BLK)])
        for c in range(BLK // C):
            _gather(c, c)
        for c in range(BLK // C):
            _gwait(c)
            pltpu.sync_copy(bufs[c], acc.at[didx.at[pl.ds(c * C, C)]],
                            add=True)

    plsc.subcore_barrier()

    @pl.when(sid < NS - 1)
    def _():
        pltpu.sync_copy(acc.at[pl.ds(sid * SEG, SEG)],
                        out_hbm.at[cid, pl.ds(sid * SEG, SEG)])

    @pl.when(sid == NS - 1)
    def _():
        pltpu.sync_copy(acc.at[pl.ds(15 * SEG, LAST_SEG)],
                        out_hbm.at[cid, pl.ds(15 * SEG, LAST_SEG)])


_agg = pl.kernel(
    _agg_body,
    out_type=jax.ShapeDtypeStruct((NC, N_NODES, D), jnp.float32),
    mesh=_sc_mesh,
    scratch_types=(
        [pltpu.VMEM((EW,), jnp.int32)] * 2
        + [pltpu.VMEM((C, D), jnp.float32)] * NB
        + [pltpu.VMEM_SHARED((N_NODES, D), jnp.float32)]
        + [pltpu.SemaphoreType.DMA] * (NB + 1)
    ),
)


def _mm1_body(feat_ref, w1_ref, o_ref):
    o_ref[...] = jnp.dot(feat_ref[...], w1_ref[...],
                         preferred_element_type=jnp.float32)


def _mm2_body(part1_ref, p_ref, w2_ref, o_ref):
    agg = p_ref[0] + p_ref[1]
    o_ref[...] = part1_ref[...] + jnp.dot(
        agg, w2_ref[...], preferred_element_type=jnp.float32)


_ROWS_BLK = 2000


def _mm1(feat, w1):
    return pl.pallas_call(
        _mm1_body,
        grid=(N_NODES // _ROWS_BLK,),
        in_specs=[
            pl.BlockSpec((_ROWS_BLK, D), lambda i: (i, 0)),
            pl.BlockSpec((D, D), lambda i: (0, 0)),
        ],
        out_specs=pl.BlockSpec((_ROWS_BLK, D), lambda i: (i, 0)),
        out_shape=jax.ShapeDtypeStruct((N_NODES, D), jnp.float32),
    )(feat, w1)


def _mm2(part1, partials, w2):
    return pl.pallas_call(
        _mm2_body,
        grid=(N_NODES // _ROWS_BLK,),
        in_specs=[
            pl.BlockSpec((_ROWS_BLK, D), lambda i: (i, 0)),
            pl.BlockSpec((NC, _ROWS_BLK, D), lambda i: (0, i, 0)),
            pl.BlockSpec((D, D), lambda i: (0, 0)),
        ],
        out_specs=pl.BlockSpec((_ROWS_BLK, D), lambda i: (i, 0)),
        out_shape=jax.ShapeDtypeStruct((N_NODES, D), jnp.float32),
    )(part1, partials, w2)


@jax.jit
def kernel(feat, edge_index, weight1, weight2):
    zeros = jnp.zeros((C, D), jnp.float32)
    partials = _agg(edge_index, feat, zeros)
    part1 = _mm1(feat, weight1)
    return _mm2(part1, partials, weight2)
